# Initial kernel scaffold; baseline (speedup 1.0000x reference)
import jax
import jax.numpy as jnp
from jax.experimental import pallas as pl


def kernel(x0, x1, x2, x3, emb_word, emb_bi, emb_tri, W1, b1, W2, b2):
    raise NotImplementedError("write your pallas kernel here")


# speedup vs baseline: 2.3078x; 2.3078x over previous
import functools

import jax
import jax.numpy as jnp
from jax import lax
from jax.experimental import pallas as pl
from jax.experimental.pallas import tpu as pltpu
from jax.experimental.pallas import tpu_sc as plsc

NC, NS, LANES = 2, 16, 16
NW = NC * NS

EMB = 64
SEQ = 200
HIDDEN = 256
NUM_LABELS = 10


def _pool_body(x0_hbm, x2_hbm, x3_hbm, ew_hbm, eb_hbm, et_hbm,
               o0_hbm, o1_hbm, o2_hbm,
               idx_v, rows_v, out_v, sem):
    batch = x0_hbm.shape[0]
    bpw = batch // NW
    wid = lax.axis_index("s") * NC + lax.axis_index("c")
    base = wid * bpw
    for xh, eh, oh in ((x0_hbm, ew_hbm, o0_hbm),
                       (x2_hbm, eb_hbm, o1_hbm),
                       (x3_hbm, et_hbm, o2_hbm)):
        @pl.loop(0, bpw)
        def _(e, xh=xh, eh=eh):
            pltpu.sync_copy(xh.at[base + e], idx_v)
            c1 = pltpu.async_copy(eh.at[idx_v.at[pl.ds(0, 128)]],
                                  rows_v.at[pl.ds(0, 128)], sem)
            c2 = pltpu.async_copy(eh.at[idx_v.at[pl.ds(128, SEQ - 128)]],
                                  rows_v.at[pl.ds(128, SEQ - 128)], sem)
            c1.wait()
            c2.wait()

            def red(s, accs):
                return tuple(a + rows_v[s, pl.ds(LANES * j, LANES)]
                             for j, a in enumerate(accs))

            accs = lax.fori_loop(
                0, SEQ, red,
                tuple(jnp.zeros((LANES,), jnp.float32)
                      for _ in range(EMB // LANES)))
            for j in range(EMB // LANES):
                out_v[e, pl.ds(LANES * j, LANES)] = accs[j]
        pltpu.sync_copy(out_v, oh.at[pl.ds(base, bpw)])


def _sc_pool(x0, x2, x3, emb_word, emb_bi, emb_tri):
    batch = x0.shape[0]
    bpw = batch // NW
    mesh = plsc.VectorSubcoreMesh(core_axis_name="c", subcore_axis_name="s")
    out = jax.ShapeDtypeStruct((batch, EMB), jnp.float32)
    return pl.kernel(
        _pool_body,
        out_type=(out, out, out),
        mesh=mesh,
        compiler_params=pltpu.CompilerParams(use_tc_tiling_on_sc=False),
        scratch_types=[
            pltpu.VMEM((SEQ,), jnp.int32),
            pltpu.VMEM((SEQ, EMB), jnp.float32),
            pltpu.VMEM((bpw, EMB), jnp.float32),
            pltpu.SemaphoreType.DMA,
        ],
    )(x0, x2, x3, emb_word, emb_bi, emb_tri)


def _mlp_body(p0_ref, p1_ref, p2_ref, w1_ref, b1_ref, w2_ref, b2_ref, o_ref):
    x = jnp.concatenate([p0_ref[...], p1_ref[...], p2_ref[...]], axis=1)
    h = jnp.dot(x, w1_ref[...], preferred_element_type=jnp.float32)
    h = h * (1.0 / SEQ) + b1_ref[...]
    h = jnp.maximum(h, 0.0)
    o_ref[...] = (jnp.dot(h, w2_ref[...], preferred_element_type=jnp.float32)
                  + b2_ref[...])


def _tc_mlp(p0, p1, p2, W1, b1, W2, b2):
    batch = p0.shape[0]
    bt = 512
    grid = (batch // bt,)
    return pl.pallas_call(
        _mlp_body,
        grid=grid,
        in_specs=[
            pl.BlockSpec((bt, EMB), lambda i: (i, 0)),
            pl.BlockSpec((bt, EMB), lambda i: (i, 0)),
            pl.BlockSpec((bt, EMB), lambda i: (i, 0)),
            pl.BlockSpec((3 * EMB, HIDDEN), lambda i: (0, 0)),
            pl.BlockSpec((1, HIDDEN), lambda i: (0, 0)),
            pl.BlockSpec((HIDDEN, NUM_LABELS), lambda i: (0, 0)),
            pl.BlockSpec((1, NUM_LABELS), lambda i: (0, 0)),
        ],
        out_specs=pl.BlockSpec((bt, NUM_LABELS), lambda i: (i, 0)),
        out_shape=jax.ShapeDtypeStruct((batch, NUM_LABELS), jnp.float32),
    )(p0, p1, p2, W1, b1, W2, b2)


def kernel(x0, x1, x2, x3, emb_word, emb_bi, emb_tri, W1, b1, W2, b2):
    del x1
    x0 = x0.astype(jnp.int32)
    p0, p1, p2 = _sc_pool(x0, x2, x3, emb_word, emb_bi, emb_tri)
    return _tc_mlp(p0, p1, p2, W1, b1.reshape(1, HIDDEN),
                   W2, b2.reshape(1, NUM_LABELS))


# speedup vs baseline: 2.9175x; 1.2642x over previous
import functools

import jax
import jax.numpy as jnp
from jax import lax
from jax.experimental import pallas as pl
from jax.experimental.pallas import tpu as pltpu
from jax.experimental.pallas import tpu_sc as plsc

NC, NS, LANES = 2, 16, 16
NW = NC * NS

EMB = 64
SEQ = 200
HIDDEN = 256
NUM_LABELS = 10


def _gather_copies(eh, idxs_v, e, rows_buf, sem):
    return (
        pltpu.make_async_copy(eh.at[idxs_v.at[e, pl.ds(0, 128)]],
                              rows_buf.at[pl.ds(0, 128)], sem),
        pltpu.make_async_copy(eh.at[idxs_v.at[e, pl.ds(128, SEQ - 128)]],
                              rows_buf.at[pl.ds(128, SEQ - 128)], sem),
    )


def _reduce_rows(rows_buf, out_v, e):
    nacc = EMB // LANES
    unroll = 8

    def red(t, accs):
        for u in range(unroll):
            s = t * unroll + u
            accs = tuple(a + rows_buf[s, pl.ds(LANES * j, LANES)]
                         for j, a in enumerate(accs))
        return accs

    accs = lax.fori_loop(
        0, SEQ // unroll, red,
        tuple(jnp.zeros((LANES,), jnp.float32) for _ in range(nacc)))
    for j in range(nacc):
        out_v[e, pl.ds(LANES * j, LANES)] = accs[j]


def _pool_body(x0_hbm, x2_hbm, x3_hbm, ew_hbm, eb_hbm, et_hbm,
               o0_hbm, o1_hbm, o2_hbm,
               idxs_v, rows_a, rows_b, out_v, sem_a, sem_b):
    batch = x0_hbm.shape[0]
    bpw = batch // NW
    wid = lax.axis_index("s") * NC + lax.axis_index("c")
    base = wid * bpw
    bufs = (rows_a, rows_b)
    sems = (sem_a, sem_b)
    for xh, eh, oh in ((x0_hbm, ew_hbm, o0_hbm),
                       (x2_hbm, eb_hbm, o1_hbm),
                       (x3_hbm, et_hbm, o2_hbm)):
        pltpu.sync_copy(xh.at[pl.ds(base, bpw)], idxs_v)
        for c in _gather_copies(eh, idxs_v, 0, bufs[0], sems[0]):
            c.start()

        @pl.loop(0, bpw // 2)
        def _(i, eh=eh):
            for p in range(2):
                e = 2 * i + p
                for c in _gather_copies(eh, idxs_v, e, bufs[p], sems[p]):
                    c.wait()
                nxt = e + 1

                @pl.when(nxt < bpw)
                def _():
                    for c in _gather_copies(eh, idxs_v, nxt,
                                            bufs[1 - p], sems[1 - p]):
                        c.start()

                _reduce_rows(bufs[p], out_v, e)

        pltpu.sync_copy(out_v, oh.at[pl.ds(base, bpw)])


def _sc_pool(x0, x2, x3, emb_word, emb_bi, emb_tri):
    batch = x0.shape[0]
    bpw = batch // NW
    mesh = plsc.VectorSubcoreMesh(core_axis_name="c", subcore_axis_name="s")
    out = jax.ShapeDtypeStruct((batch, EMB), jnp.float32)
    return pl.kernel(
        _pool_body,
        out_type=(out, out, out),
        mesh=mesh,
        compiler_params=pltpu.CompilerParams(use_tc_tiling_on_sc=False),
        scratch_types=[
            pltpu.VMEM((bpw, SEQ), jnp.int32),
            pltpu.VMEM((SEQ, EMB), jnp.float32),
            pltpu.VMEM((SEQ, EMB), jnp.float32),
            pltpu.VMEM((bpw, EMB), jnp.float32),
            pltpu.SemaphoreType.DMA,
            pltpu.SemaphoreType.DMA,
        ],
    )(x0, x2, x3, emb_word, emb_bi, emb_tri)


def _mlp_body(p0_ref, p1_ref, p2_ref, w1_ref, b1_ref, w2_ref, b2_ref, o_ref):
    x = jnp.concatenate([p0_ref[...], p1_ref[...], p2_ref[...]], axis=1)
    h = jnp.dot(x, w1_ref[...], preferred_element_type=jnp.float32)
    h = h * (1.0 / SEQ) + b1_ref[...]
    h = jnp.maximum(h, 0.0)
    o_ref[...] = (jnp.dot(h, w2_ref[...], preferred_element_type=jnp.float32)
                  + b2_ref[...])


def _tc_mlp(p0, p1, p2, W1, b1, W2, b2):
    batch = p0.shape[0]
    bt = 512
    grid = (batch // bt,)
    return pl.pallas_call(
        _mlp_body,
        grid=grid,
        in_specs=[
            pl.BlockSpec((bt, EMB), lambda i: (i, 0)),
            pl.BlockSpec((bt, EMB), lambda i: (i, 0)),
            pl.BlockSpec((bt, EMB), lambda i: (i, 0)),
            pl.BlockSpec((3 * EMB, HIDDEN), lambda i: (0, 0)),
            pl.BlockSpec((1, HIDDEN), lambda i: (0, 0)),
            pl.BlockSpec((HIDDEN, NUM_LABELS), lambda i: (0, 0)),
            pl.BlockSpec((1, NUM_LABELS), lambda i: (0, 0)),
        ],
        out_specs=pl.BlockSpec((bt, NUM_LABELS), lambda i: (i, 0)),
        out_shape=jax.ShapeDtypeStruct((batch, NUM_LABELS), jnp.float32),
    )(p0, p1, p2, W1, b1, W2, b2)


def kernel(x0, x1, x2, x3, emb_word, emb_bi, emb_tri, W1, b1, W2, b2):
    del x1
    x0 = x0.astype(jnp.int32)
    p0, p1, p2 = _sc_pool(x0, x2, x3, emb_word, emb_bi, emb_tri)
    return _tc_mlp(p0, p1, p2, W1, b1.reshape(1, HIDDEN),
                   W2, b2.reshape(1, NUM_LABELS))
